# Initial kernel scaffold; baseline (speedup 1.0000x reference)
import jax
import jax.numpy as jnp
from jax.experimental import pallas as pl


def kernel(x, cpd_centroid, y, uvp_dim, sigma, enc_nW, enc_nb, enc_eW, enc_eb, eW0, eb0, nW0, nb0, eW1, eb1, nW1, nb1, dec_W, dec_b, edge_index, batch, cell_type):
    raise NotImplementedError("write your pallas kernel here")


# speedup vs baseline: 2.1871x; 2.1871x over previous
import functools

import jax
import jax.numpy as jnp
from jax import lax
from jax.experimental import pallas as pl
from jax.experimental.pallas import tpu as pltpu
from jax.experimental.pallas import tpu_sc as plsc

_N, _E, _B = 50000, 800000, 4
_NI, _PHI, _H = 7, 3, 64

_RN = 2000
_GN = _N // _RN
_RE = 2048
_EPAD = 802816
_GE = _EPAD // _RE
_CH = 128
_EPW = _EPAD // 32
_EPS = _EPAD // 16
_NACC = 51200
_RPT = _NACC // 16


def _stats_body(x_ref, b_ref, o_ref, acc_ref):
    i = pl.program_id(0)

    @pl.when(i == 0)
    def _():
        acc_ref[...] = jnp.zeros_like(acc_ref)

    xb = x_ref[...]
    bb = b_ref[...]
    feat = jnp.concatenate(
        [xb, jnp.ones((_RN, 1), jnp.float32), xb * xb,
         jnp.zeros((_RN, 1), jnp.float32)], axis=1)
    for b in range(_B):
        m = (bb == b).astype(jnp.float32)
        s = jnp.sum(feat * m, axis=0, keepdims=True)
        acc_ref[pl.ds(b, 1), :] = acc_ref[pl.ds(b, 1), :] + s

    @pl.when(i == _GN - 1)
    def _():
        A = acc_ref[...]
        cnt = jnp.maximum(A[0:4, 7:8], 1.0)
        mean = A[0:4, 0:7] / cnt
        ex2 = A[0:4, 8:15] / cnt
        var = ex2 - mean * mean
        gsum = jnp.sum(A[0:4, 3:7], axis=0, keepdims=True)
        gsq = jnp.sum(A[0:4, 11:15], axis=0, keepdims=True)
        gmean = gsum / float(_N)
        gvar = gsq / float(_N) - gmean * gmean
        sc_phi = 1.0 / (jnp.sqrt(jnp.maximum(var[:, 0:3], 0.0)) + 1e-8)
        sc_g = 1.0 / (jnp.sqrt(jnp.maximum(gvar, 0.0)) + 1e-8)
        sc_gb = jnp.broadcast_to(sc_g, (4, 4))
        sh_phi = -mean[:, 0:3] * sc_phi
        sh_g = jnp.broadcast_to(-gmean * sc_g, (4, 4))
        scale = jnp.concatenate(
            [sc_phi, sc_gb, jnp.ones((4, 1), jnp.float32)], axis=1)
        shift = jnp.concatenate(
            [sh_phi, sh_g, jnp.zeros((4, 1), jnp.float32)], axis=1)
        o_ref[...] = jnp.concatenate([scale, shift], axis=0)


def _sel4(bb, rows):
    return jnp.where(
        bb == 0, rows[0:1],
        jnp.where(bb == 1, rows[1:2],
                  jnp.where(bb == 2, rows[2:3], rows[3:4])))


def _encode_body(x_ref, c_ref, b_ref, st_ref, w_ref, bias_ref, xnc_ref, h_ref):
    xb = x_ref[...]
    cb = c_ref[...]
    bb = b_ref[...]
    st = st_ref[...]
    sc = _sel4(bb, st[0:4])
    sh = _sel4(bb, st[4:8])
    xn = xb * sc[:, 0:7] + sh[:, 0:7]
    xnc_ref[...] = jnp.concatenate(
        [xn, cb, jnp.zeros((_RN, 7), jnp.float32)], axis=1)
    xn8 = jnp.concatenate([xn, jnp.zeros((_RN, 1), jnp.float32)], axis=1)
    h_ref[...] = jnp.maximum(
        jnp.dot(xn8, w_ref[...], preferred_element_type=jnp.float32)
        + bias_ref[...], 0.0)


def _edge_enc_body(s_ref, r_ref, w_ref, bias_ref, e_ref):
    d = s_ref[...] - r_ref[...]
    nrm = jnp.sqrt(d[:, 7:8] * d[:, 7:8] + d[:, 8:9] * d[:, 8:9])
    ea = jnp.concatenate(
        [d[:, 0:9], nrm, jnp.zeros((_RE, 6), jnp.float32)], axis=1)
    e_ref[...] = jnp.maximum(
        jnp.dot(ea, w_ref[...], preferred_element_type=jnp.float32)
        + bias_ref[...], 0.0)


def _msg_body(e_ref, hs_ref, hr_ref, we_ref, ws_ref, wr_ref, bias_ref,
              enew_ref, mlo_ref, mhi_ref):
    e = e_ref[...]
    m = (jnp.dot(e, we_ref[...], preferred_element_type=jnp.float32)
         + jnp.dot(hs_ref[...], ws_ref[...], preferred_element_type=jnp.float32)
         + jnp.dot(hr_ref[...], wr_ref[...], preferred_element_type=jnp.float32)
         + bias_ref[...])
    m = jnp.maximum(m, 0.0)
    enew_ref[...] = e + m
    mlo_ref[...] = m[:, 0:32]
    mhi_ref[...] = m[:, 32:64]


def _node_upd_body(h_ref, alo_ref, ahi_ref, wh_ref, wlo_ref, whi_ref,
                   bias_ref, out_ref):
    h = h_ref[...]
    u = (jnp.dot(h, wh_ref[...], preferred_element_type=jnp.float32)
         + jnp.dot(alo_ref[...], wlo_ref[...], preferred_element_type=jnp.float32)
         + jnp.dot(ahi_ref[...], whi_ref[...], preferred_element_type=jnp.float32)
         + bias_ref[...])
    out_ref[...] = h + jnp.maximum(u, 0.0)


def _decode_body(h_ref, y_ref, ct_ref, b_ref, wd_ref, bd_ref, f_ref, out_ref):
    h = h_ref[...]
    uvp = (jnp.dot(h, wd_ref[...], preferred_element_type=jnp.float32)
           + bd_ref[...])
    ct = ct_ref[...]
    bb = b_ref[...]
    mask_d = (ct == 6) | (ct == 4) | (ct == 7) | (ct == 8)
    mask_p = ct == 7
    yb = y_ref[...]
    uv = jnp.where(mask_d, yb[:, 0:2], uvp[:, 0:2])
    p = jnp.where(mask_p, 0.0, uvp[:, 2:3])
    fac = _sel4(bb, f_ref[...][0:4])
    out_ref[...] = jnp.concatenate([uv, p], axis=1) * fac[:, 0:3]


def _make_gather(d):
    mesh = plsc.VectorSubcoreMesh(core_axis_name="c", subcore_axis_name="s")

    @functools.partial(
        pl.kernel,
        out_type=[jax.ShapeDtypeStruct((_EPAD, d), jnp.float32),
                  jax.ShapeDtypeStruct((_EPAD, d), jnp.float32)],
        mesh=mesh,
        scratch_types=[pltpu.VMEM((_CH,), jnp.int32),
                       pltpu.VMEM((_CH, d), jnp.float32),
                       pltpu.SemaphoreType.DMA],
        compiler_params=pltpu.CompilerParams(use_tc_tiling_on_sc=False),
    )
    def gat(tab, si, ri, out_s, out_r, idx_v, rows_v, sem):
        wid = lax.axis_index("s") * 2 + lax.axis_index("c")
        base = wid * _EPW

        def body(j, carry):
            off = base + j * _CH
            pltpu.sync_copy(si.at[pl.ds(off, _CH)], idx_v)
            pltpu.async_copy(tab.at[idx_v], rows_v, sem).wait()
            pltpu.sync_copy(rows_v, out_s.at[pl.ds(off, _CH)])
            pltpu.sync_copy(ri.at[pl.ds(off, _CH)], idx_v)
            pltpu.async_copy(tab.at[idx_v], rows_v, sem).wait()
            pltpu.sync_copy(rows_v, out_r.at[pl.ds(off, _CH)])
            return carry

        lax.fori_loop(0, _EPW // _CH, body, 0)

    return gat


_gather16 = _make_gather(16)
_gather64 = _make_gather(64)


_scatter_mesh = plsc.VectorSubcoreMesh(core_axis_name="c", subcore_axis_name="s")


@functools.partial(
    pl.kernel,
    out_type=[jax.ShapeDtypeStruct((_NACC, 32), jnp.float32),
              jax.ShapeDtypeStruct((_NACC, 32), jnp.float32)],
    mesh=_scatter_mesh,
    scratch_types=[pltpu.VMEM_SHARED((_NACC, 32), jnp.float32),
                   pltpu.VMEM((_CH,), jnp.int32),
                   pltpu.VMEM((_CH, 32), jnp.float32),
                   pltpu.VMEM((_CH, 32), jnp.float32)],
    compiler_params=pltpu.CompilerParams(use_tc_tiling_on_sc=False),
)
def _scatter(ridx, mlo, mhi, alo, ahi, acc, idx_v, rows_v, zbuf):
    c = lax.axis_index("c")
    s = lax.axis_index("s")
    z16 = jnp.zeros((16,), jnp.float32)

    def zrow(i, carry):
        zbuf[i, pl.ds(0, 16)] = z16
        zbuf[i, pl.ds(16, 16)] = z16
        return carry

    lax.fori_loop(0, _CH, zrow, 0)
    rbase = s * _RPT

    def zc(k, carry):
        pltpu.sync_copy(zbuf, acc.at[pl.ds(rbase + k * _CH, _CH)])
        return carry

    lax.fori_loop(0, _RPT // _CH, zc, 0)
    plsc.subcore_barrier()

    ebase = s * _EPS

    def scatter_from(m_ref):
        def body(j, carry):
            off = ebase + j * _CH
            pltpu.sync_copy(ridx.at[pl.ds(off, _CH)], idx_v)
            pltpu.sync_copy(m_ref.at[pl.ds(off, _CH)], rows_v)
            pltpu.sync_copy(rows_v, acc.at[idx_v], add=True)
            return carry

        lax.fori_loop(0, _EPS // _CH, body, 0)

    @pl.when(c == 0)
    def _():
        scatter_from(mlo)

    @pl.when(c == 1)
    def _():
        scatter_from(mhi)

    plsc.subcore_barrier()

    def wo(k, carry):
        r = rbase + k * _CH
        pltpu.sync_copy(acc.at[pl.ds(r, _CH)], rows_v)

        @pl.when(c == 0)
        def _():
            pltpu.sync_copy(rows_v, alo.at[pl.ds(r, _CH)])

        @pl.when(c == 1)
        def _():
            pltpu.sync_copy(rows_v, ahi.at[pl.ds(r, _CH)])

        return carry

    lax.fori_loop(0, _RPT // _CH, wo, 0)


def _full(shape):
    zeros = (0,) * len(shape)
    return pl.BlockSpec(shape, lambda i: zeros)


def kernel(x, cpd_centroid, y, uvp_dim, sigma, enc_nW, enc_nb, enc_eW, enc_eb,
           eW0, eb0, nW0, nb0, eW1, eb1, nW1, nb1, dec_W, dec_b,
           edge_index, batch, cell_type):
    f32 = jnp.float32
    batch2 = batch.reshape(_N, 1)
    ct2 = cell_type.reshape(_N, 1)
    s_idx = edge_index[0]
    r_idx = edge_index[1]
    pad = _EPAD - _E
    sg = jnp.concatenate([s_idx, jnp.zeros((pad,), jnp.int32)])
    rg = jnp.concatenate([r_idx, jnp.zeros((pad,), jnp.int32)])
    rs = jnp.concatenate([r_idx, jnp.full((pad,), _N, jnp.int32)])

    wn8 = jnp.concatenate([enc_nW, jnp.zeros((1, _H), f32)], axis=0)
    bn = enc_nb.reshape(1, _H)
    we16 = jnp.concatenate([enc_eW, jnp.zeros((6, _H), f32)], axis=0)
    be = enc_eb.reshape(1, _H)
    wd8 = jnp.concatenate([dec_W, jnp.zeros((_H, 5), f32)], axis=1)
    bd8 = jnp.concatenate([dec_b, jnp.zeros((5,), f32)]).reshape(1, 8)
    fac = uvp_dim * sigma
    fac8 = jnp.zeros((8, 8), f32).at[0:4, 0:3].set(fac)

    st = pl.pallas_call(
        _stats_body,
        grid=(_GN,),
        in_specs=[pl.BlockSpec((_RN, _NI), lambda i: (i, 0)),
                  pl.BlockSpec((_RN, 1), lambda i: (i, 0))],
        out_specs=_full((8, 8)),
        out_shape=jax.ShapeDtypeStruct((8, 8), f32),
        scratch_shapes=[pltpu.VMEM((8, 16), f32)],
    )(x, batch2)

    xnc, h = pl.pallas_call(
        _encode_body,
        grid=(_GN,),
        in_specs=[pl.BlockSpec((_RN, _NI), lambda i: (i, 0)),
                  pl.BlockSpec((_RN, 2), lambda i: (i, 0)),
                  pl.BlockSpec((_RN, 1), lambda i: (i, 0)),
                  _full((8, 8)), _full((8, _H)), _full((1, _H))],
        out_specs=[pl.BlockSpec((_RN, 16), lambda i: (i, 0)),
                   pl.BlockSpec((_RN, _H), lambda i: (i, 0))],
        out_shape=[jax.ShapeDtypeStruct((_N, 16), f32),
                   jax.ShapeDtypeStruct((_N, _H), f32)],
    )(x, cpd_centroid, batch2, st, wn8, bn)

    xs, xr = _gather16(xnc, sg, rg)
    e = pl.pallas_call(
        _edge_enc_body,
        grid=(_GE,),
        in_specs=[pl.BlockSpec((_RE, 16), lambda i: (i, 0)),
                  pl.BlockSpec((_RE, 16), lambda i: (i, 0)),
                  _full((16, _H)), _full((1, _H))],
        out_specs=pl.BlockSpec((_RE, _H), lambda i: (i, 0)),
        out_shape=jax.ShapeDtypeStruct((_EPAD, _H), f32),
    )(xs, xr, we16, be)

    for eW, eb, nW, nb in ((eW0, eb0, nW0, nb0), (eW1, eb1, nW1, nb1)):
        hs, hr = _gather64(h, sg, rg)
        e, mlo, mhi = pl.pallas_call(
            _msg_body,
            grid=(_GE,),
            in_specs=[pl.BlockSpec((_RE, _H), lambda i: (i, 0)),
                      pl.BlockSpec((_RE, _H), lambda i: (i, 0)),
                      pl.BlockSpec((_RE, _H), lambda i: (i, 0)),
                      _full((_H, _H)), _full((_H, _H)), _full((_H, _H)),
                      _full((1, _H))],
            out_specs=[pl.BlockSpec((_RE, _H), lambda i: (i, 0)),
                       pl.BlockSpec((_RE, 32), lambda i: (i, 0)),
                       pl.BlockSpec((_RE, 32), lambda i: (i, 0))],
            out_shape=[jax.ShapeDtypeStruct((_EPAD, _H), f32),
                       jax.ShapeDtypeStruct((_EPAD, 32), f32),
                       jax.ShapeDtypeStruct((_EPAD, 32), f32)],
        )(e, hs, hr, eW[0:_H], eW[_H:2 * _H], eW[2 * _H:3 * _H],
          eb.reshape(1, _H))
        alo, ahi = _scatter(rs, mlo, mhi)
        h = pl.pallas_call(
            _node_upd_body,
            grid=(_GN,),
            in_specs=[pl.BlockSpec((_RN, _H), lambda i: (i, 0)),
                      pl.BlockSpec((_RN, 32), lambda i: (i, 0)),
                      pl.BlockSpec((_RN, 32), lambda i: (i, 0)),
                      _full((_H, _H)), _full((32, _H)), _full((32, _H)),
                      _full((1, _H))],
            out_specs=pl.BlockSpec((_RN, _H), lambda i: (i, 0)),
            out_shape=jax.ShapeDtypeStruct((_N, _H), f32),
        )(h, alo, ahi, nW[0:_H], nW[_H:_H + 32], nW[_H + 32:_H + 64],
          nb.reshape(1, _H))

    out = pl.pallas_call(
        _decode_body,
        grid=(_GN,),
        in_specs=[pl.BlockSpec((_RN, _H), lambda i: (i, 0)),
                  pl.BlockSpec((_RN, 3), lambda i: (i, 0)),
                  pl.BlockSpec((_RN, 1), lambda i: (i, 0)),
                  pl.BlockSpec((_RN, 1), lambda i: (i, 0)),
                  _full((_H, 8)), _full((1, 8)), _full((8, 8))],
        out_specs=pl.BlockSpec((_RN, 3), lambda i: (i, 0)),
        out_shape=jax.ShapeDtypeStruct((_N, 3), f32),
    )(h, y, ct2, batch2, wd8, bd8, fac8)
    return out


# speedup vs baseline: 2.5793x; 1.1793x over previous
import functools

import jax
import jax.numpy as jnp
from jax import lax
from jax.experimental import pallas as pl
from jax.experimental.pallas import tpu as pltpu
from jax.experimental.pallas import tpu_sc as plsc

_N, _E, _B = 50000, 800000, 4
_NI, _PHI, _H = 7, 3, 64

_RN = 2000
_GN = _N // _RN
_RE = 2048
_EPAD = 802816
_GE = _EPAD // _RE
_CH = 128
_EPW = _EPAD // 32
_EPS = _EPAD // 16
_NACC = 51200
_RPT = _NACC // 16


def _stats_body(x_ref, b_ref, o_ref, acc_ref):
    i = pl.program_id(0)

    @pl.when(i == 0)
    def _():
        acc_ref[...] = jnp.zeros_like(acc_ref)

    xb = x_ref[...]
    bb = b_ref[...]
    feat = jnp.concatenate(
        [xb, jnp.ones((_RN, 1), jnp.float32), xb * xb,
         jnp.zeros((_RN, 1), jnp.float32)], axis=1)
    for b in range(_B):
        m = (bb == b).astype(jnp.float32)
        s = jnp.sum(feat * m, axis=0, keepdims=True)
        acc_ref[pl.ds(b, 1), :] = acc_ref[pl.ds(b, 1), :] + s

    @pl.when(i == _GN - 1)
    def _():
        A = acc_ref[...]
        cnt = jnp.maximum(A[0:4, 7:8], 1.0)
        mean = A[0:4, 0:7] / cnt
        ex2 = A[0:4, 8:15] / cnt
        var = ex2 - mean * mean
        gsum = jnp.sum(A[0:4, 3:7], axis=0, keepdims=True)
        gsq = jnp.sum(A[0:4, 11:15], axis=0, keepdims=True)
        gmean = gsum / float(_N)
        gvar = gsq / float(_N) - gmean * gmean
        sc_phi = 1.0 / (jnp.sqrt(jnp.maximum(var[:, 0:3], 0.0)) + 1e-8)
        sc_g = 1.0 / (jnp.sqrt(jnp.maximum(gvar, 0.0)) + 1e-8)
        sc_gb = jnp.broadcast_to(sc_g, (4, 4))
        sh_phi = -mean[:, 0:3] * sc_phi
        sh_g = jnp.broadcast_to(-gmean * sc_g, (4, 4))
        scale = jnp.concatenate(
            [sc_phi, sc_gb, jnp.ones((4, 1), jnp.float32)], axis=1)
        shift = jnp.concatenate(
            [sh_phi, sh_g, jnp.zeros((4, 1), jnp.float32)], axis=1)
        o_ref[...] = jnp.concatenate([scale, shift], axis=0)


def _sel4(bb, rows):
    return jnp.where(
        bb == 0, rows[0:1],
        jnp.where(bb == 1, rows[1:2],
                  jnp.where(bb == 2, rows[2:3], rows[3:4])))


def _encode_body(x_ref, c_ref, b_ref, st_ref, w_ref, bias_ref, xnc_ref, h_ref):
    xb = x_ref[...]
    cb = c_ref[...]
    bb = b_ref[...]
    st = st_ref[...]
    sc = _sel4(bb, st[0:4])
    sh = _sel4(bb, st[4:8])
    xn = xb * sc[:, 0:7] + sh[:, 0:7]
    xnc_ref[...] = jnp.concatenate(
        [xn, cb, jnp.zeros((_RN, 7), jnp.float32)], axis=1)
    xn8 = jnp.concatenate([xn, jnp.zeros((_RN, 1), jnp.float32)], axis=1)
    h_ref[...] = jnp.maximum(
        jnp.dot(xn8, w_ref[...], preferred_element_type=jnp.float32)
        + bias_ref[...], 0.0)


def _edge_enc_body(s_ref, r_ref, w_ref, bias_ref, e_ref):
    d = s_ref[...] - r_ref[...]
    nrm = jnp.sqrt(d[:, 7:8] * d[:, 7:8] + d[:, 8:9] * d[:, 8:9])
    ea = jnp.concatenate(
        [d[:, 0:9], nrm, jnp.zeros((_RE, 6), jnp.float32)], axis=1)
    e_ref[...] = jnp.maximum(
        jnp.dot(ea, w_ref[...], preferred_element_type=jnp.float32)
        + bias_ref[...], 0.0)


def _msg_body(e_ref, hs_ref, hr_ref, we_ref, ws_ref, wr_ref, bias_ref,
              enew_ref, mlo_ref, mhi_ref):
    e = e_ref[...]
    m = (jnp.dot(e, we_ref[...], preferred_element_type=jnp.float32)
         + jnp.dot(hs_ref[...], ws_ref[...], preferred_element_type=jnp.float32)
         + jnp.dot(hr_ref[...], wr_ref[...], preferred_element_type=jnp.float32)
         + bias_ref[...])
    m = jnp.maximum(m, 0.0)
    enew_ref[...] = e + m
    mlo_ref[...] = m[:, 0:32]
    mhi_ref[...] = m[:, 32:64]


def _node_upd_body(h_ref, alo_ref, ahi_ref, wh_ref, wlo_ref, whi_ref,
                   bias_ref, out_ref):
    h = h_ref[...]
    u = (jnp.dot(h, wh_ref[...], preferred_element_type=jnp.float32)
         + jnp.dot(alo_ref[...], wlo_ref[...], preferred_element_type=jnp.float32)
         + jnp.dot(ahi_ref[...], whi_ref[...], preferred_element_type=jnp.float32)
         + bias_ref[...])
    out_ref[...] = h + jnp.maximum(u, 0.0)


def _decode_body(h_ref, y_ref, ct_ref, b_ref, wd_ref, bd_ref, f_ref, out_ref):
    h = h_ref[...]
    uvp = (jnp.dot(h, wd_ref[...], preferred_element_type=jnp.float32)
           + bd_ref[...])
    ct = ct_ref[...]
    bb = b_ref[...]
    mask_d = (ct == 6) | (ct == 4) | (ct == 7) | (ct == 8)
    mask_p = ct == 7
    yb = y_ref[...]
    uv = jnp.where(mask_d, yb[:, 0:2], uvp[:, 0:2])
    p = jnp.where(mask_p, 0.0, uvp[:, 2:3])
    fac = _sel4(bb, f_ref[...][0:4])
    out_ref[...] = jnp.concatenate([uv, p], axis=1) * fac[:, 0:3]


def _make_gather(d):
    mesh = plsc.VectorSubcoreMesh(core_axis_name="c", subcore_axis_name="s")

    @functools.partial(
        pl.kernel,
        out_type=[jax.ShapeDtypeStruct((_EPAD, d), jnp.float32),
                  jax.ShapeDtypeStruct((_EPAD, d), jnp.float32)],
        mesh=mesh,
        scratch_types=[pltpu.VMEM((_CH,), jnp.int32),
                       pltpu.VMEM((_CH,), jnp.int32),
                       pltpu.VMEM((_CH,), jnp.int32),
                       pltpu.VMEM((_CH,), jnp.int32),
                       pltpu.VMEM((_CH, d), jnp.float32),
                       pltpu.VMEM((_CH, d), jnp.float32),
                       pltpu.VMEM((_CH, d), jnp.float32),
                       pltpu.VMEM((_CH, d), jnp.float32)]
        + [pltpu.SemaphoreType.DMA] * 8,
        compiler_params=pltpu.CompilerParams(use_tc_tiling_on_sc=False),
    )
    def gat(tab, si, ri, out_s, out_r, ia0, ir0, ia1, ir1,
            ra0, rr0, ra1, rr1, g0, g1, g2, g3, t0, t1, t2, t3):
        wid = lax.axis_index("s") * 2 + lax.axis_index("c")
        base = wid * _EPW

        def body(j, carry):
            offa = base + 2 * j * _CH
            offb = offa + _CH
            pltpu.sync_copy(si.at[pl.ds(offa, _CH)], ia0)
            pltpu.sync_copy(ri.at[pl.ds(offa, _CH)], ir0)
            pltpu.sync_copy(si.at[pl.ds(offb, _CH)], ia1)
            pltpu.sync_copy(ri.at[pl.ds(offb, _CH)], ir1)
            ga0 = pltpu.async_copy(tab.at[ia0], ra0, g0)
            gr0 = pltpu.async_copy(tab.at[ir0], rr0, g1)
            ga1 = pltpu.async_copy(tab.at[ia1], ra1, g2)
            gr1 = pltpu.async_copy(tab.at[ir1], rr1, g3)
            ga0.wait()
            s0 = pltpu.async_copy(ra0, out_s.at[pl.ds(offa, _CH)], t0)
            gr0.wait()
            s1 = pltpu.async_copy(rr0, out_r.at[pl.ds(offa, _CH)], t1)
            ga1.wait()
            s2 = pltpu.async_copy(ra1, out_s.at[pl.ds(offb, _CH)], t2)
            gr1.wait()
            s3 = pltpu.async_copy(rr1, out_r.at[pl.ds(offb, _CH)], t3)
            s0.wait()
            s1.wait()
            s2.wait()
            s3.wait()
            return carry

        lax.fori_loop(0, _EPW // (2 * _CH), body, 0)

    return gat


_gather16 = _make_gather(16)
_gather64 = _make_gather(64)


_scatter_mesh = plsc.VectorSubcoreMesh(core_axis_name="c", subcore_axis_name="s")


@functools.partial(
    pl.kernel,
    out_type=[jax.ShapeDtypeStruct((_NACC, 32), jnp.float32),
              jax.ShapeDtypeStruct((_NACC, 32), jnp.float32)],
    mesh=_scatter_mesh,
    scratch_types=[pltpu.VMEM_SHARED((_NACC, 32), jnp.float32),
                   pltpu.VMEM((_CH,), jnp.int32),
                   pltpu.VMEM((_CH,), jnp.int32),
                   pltpu.VMEM((_CH, 32), jnp.float32),
                   pltpu.VMEM((_CH, 32), jnp.float32),
                   pltpu.VMEM((_CH, 32), jnp.float32)]
    + [pltpu.SemaphoreType.DMA] * 6,
    compiler_params=pltpu.CompilerParams(use_tc_tiling_on_sc=False),
)
def _scatter(ridx, mlo, mhi, alo, ahi, acc, i0, i1, m0, m1, zbuf,
             si0, si1, sm0, sm1, sa0, sa1):
    c = lax.axis_index("c")
    s = lax.axis_index("s")
    z16 = jnp.zeros((16,), jnp.float32)

    def zrow(i, carry):
        zbuf[i, pl.ds(0, 16)] = z16
        zbuf[i, pl.ds(16, 16)] = z16
        return carry

    lax.fori_loop(0, _CH, zrow, 0)
    rbase = s * _RPT

    def zc(k, carry):
        pltpu.sync_copy(zbuf, acc.at[pl.ds(rbase + k * _CH, _CH)])
        return carry

    lax.fori_loop(0, _RPT // _CH, zc, 0)
    plsc.subcore_barrier()

    ebase = s * _EPS

    def scatter_from(m_ref):
        def body(j, carry):
            offa = ebase + 2 * j * _CH
            offb = offa + _CH
            ci0 = pltpu.async_copy(ridx.at[pl.ds(offa, _CH)], i0, si0)
            cm0 = pltpu.async_copy(m_ref.at[pl.ds(offa, _CH)], m0, sm0)
            ci1 = pltpu.async_copy(ridx.at[pl.ds(offb, _CH)], i1, si1)
            cm1 = pltpu.async_copy(m_ref.at[pl.ds(offb, _CH)], m1, sm1)
            ci0.wait()
            cm0.wait()
            a0 = pltpu.async_copy(m0, acc.at[i0], sa0, add=True)
            ci1.wait()
            cm1.wait()
            a1 = pltpu.async_copy(m1, acc.at[i1], sa1, add=True)
            a0.wait()
            a1.wait()
            return carry

        lax.fori_loop(0, _EPS // (2 * _CH), body, 0)

    @pl.when(c == 0)
    def _():
        scatter_from(mlo)

    @pl.when(c == 1)
    def _():
        scatter_from(mhi)

    plsc.subcore_barrier()

    def wo(k, carry):
        r = rbase + k * _CH
        pltpu.sync_copy(acc.at[pl.ds(r, _CH)], m0)

        @pl.when(c == 0)
        def _():
            pltpu.sync_copy(m0, alo.at[pl.ds(r, _CH)])

        @pl.when(c == 1)
        def _():
            pltpu.sync_copy(m0, ahi.at[pl.ds(r, _CH)])

        return carry

    lax.fori_loop(0, _RPT // _CH, wo, 0)


def _full(shape):
    zeros = (0,) * len(shape)
    return pl.BlockSpec(shape, lambda i: zeros)


def kernel(x, cpd_centroid, y, uvp_dim, sigma, enc_nW, enc_nb, enc_eW, enc_eb,
           eW0, eb0, nW0, nb0, eW1, eb1, nW1, nb1, dec_W, dec_b,
           edge_index, batch, cell_type):
    f32 = jnp.float32
    batch2 = batch.reshape(_N, 1)
    ct2 = cell_type.reshape(_N, 1)
    s_idx = edge_index[0]
    r_idx = edge_index[1]
    pad = _EPAD - _E
    sg = jnp.concatenate([s_idx, jnp.zeros((pad,), jnp.int32)])
    rg = jnp.concatenate([r_idx, jnp.zeros((pad,), jnp.int32)])
    rs = jnp.concatenate([r_idx, jnp.full((pad,), _N, jnp.int32)])

    wn8 = jnp.concatenate([enc_nW, jnp.zeros((1, _H), f32)], axis=0)
    bn = enc_nb.reshape(1, _H)
    we16 = jnp.concatenate([enc_eW, jnp.zeros((6, _H), f32)], axis=0)
    be = enc_eb.reshape(1, _H)
    wd8 = jnp.concatenate([dec_W, jnp.zeros((_H, 5), f32)], axis=1)
    bd8 = jnp.concatenate([dec_b, jnp.zeros((5,), f32)]).reshape(1, 8)
    fac = uvp_dim * sigma
    fac8 = jnp.zeros((8, 8), f32).at[0:4, 0:3].set(fac)

    st = pl.pallas_call(
        _stats_body,
        grid=(_GN,),
        in_specs=[pl.BlockSpec((_RN, _NI), lambda i: (i, 0)),
                  pl.BlockSpec((_RN, 1), lambda i: (i, 0))],
        out_specs=_full((8, 8)),
        out_shape=jax.ShapeDtypeStruct((8, 8), f32),
        scratch_shapes=[pltpu.VMEM((8, 16), f32)],
    )(x, batch2)

    xnc, h = pl.pallas_call(
        _encode_body,
        grid=(_GN,),
        in_specs=[pl.BlockSpec((_RN, _NI), lambda i: (i, 0)),
                  pl.BlockSpec((_RN, 2), lambda i: (i, 0)),
                  pl.BlockSpec((_RN, 1), lambda i: (i, 0)),
                  _full((8, 8)), _full((8, _H)), _full((1, _H))],
        out_specs=[pl.BlockSpec((_RN, 16), lambda i: (i, 0)),
                   pl.BlockSpec((_RN, _H), lambda i: (i, 0))],
        out_shape=[jax.ShapeDtypeStruct((_N, 16), f32),
                   jax.ShapeDtypeStruct((_N, _H), f32)],
    )(x, cpd_centroid, batch2, st, wn8, bn)

    xs, xr = _gather16(xnc, sg, rg)
    e = pl.pallas_call(
        _edge_enc_body,
        grid=(_GE,),
        in_specs=[pl.BlockSpec((_RE, 16), lambda i: (i, 0)),
                  pl.BlockSpec((_RE, 16), lambda i: (i, 0)),
                  _full((16, _H)), _full((1, _H))],
        out_specs=pl.BlockSpec((_RE, _H), lambda i: (i, 0)),
        out_shape=jax.ShapeDtypeStruct((_EPAD, _H), f32),
    )(xs, xr, we16, be)

    for eW, eb, nW, nb in ((eW0, eb0, nW0, nb0), (eW1, eb1, nW1, nb1)):
        hs, hr = _gather64(h, sg, rg)
        e, mlo, mhi = pl.pallas_call(
            _msg_body,
            grid=(_GE,),
            in_specs=[pl.BlockSpec((_RE, _H), lambda i: (i, 0)),
                      pl.BlockSpec((_RE, _H), lambda i: (i, 0)),
                      pl.BlockSpec((_RE, _H), lambda i: (i, 0)),
                      _full((_H, _H)), _full((_H, _H)), _full((_H, _H)),
                      _full((1, _H))],
            out_specs=[pl.BlockSpec((_RE, _H), lambda i: (i, 0)),
                       pl.BlockSpec((_RE, 32), lambda i: (i, 0)),
                       pl.BlockSpec((_RE, 32), lambda i: (i, 0))],
            out_shape=[jax.ShapeDtypeStruct((_EPAD, _H), f32),
                       jax.ShapeDtypeStruct((_EPAD, 32), f32),
                       jax.ShapeDtypeStruct((_EPAD, 32), f32)],
        )(e, hs, hr, eW[0:_H], eW[_H:2 * _H], eW[2 * _H:3 * _H],
          eb.reshape(1, _H))
        alo, ahi = _scatter(rs, mlo, mhi)
        h = pl.pallas_call(
            _node_upd_body,
            grid=(_GN,),
            in_specs=[pl.BlockSpec((_RN, _H), lambda i: (i, 0)),
                      pl.BlockSpec((_RN, 32), lambda i: (i, 0)),
                      pl.BlockSpec((_RN, 32), lambda i: (i, 0)),
                      _full((_H, _H)), _full((32, _H)), _full((32, _H)),
                      _full((1, _H))],
            out_specs=pl.BlockSpec((_RN, _H), lambda i: (i, 0)),
            out_shape=jax.ShapeDtypeStruct((_N, _H), f32),
        )(h, alo, ahi, nW[0:_H], nW[_H:_H + 32], nW[_H + 32:_H + 64],
          nb.reshape(1, _H))

    out = pl.pallas_call(
        _decode_body,
        grid=(_GN,),
        in_specs=[pl.BlockSpec((_RN, _H), lambda i: (i, 0)),
                  pl.BlockSpec((_RN, 3), lambda i: (i, 0)),
                  pl.BlockSpec((_RN, 1), lambda i: (i, 0)),
                  pl.BlockSpec((_RN, 1), lambda i: (i, 0)),
                  _full((_H, 8)), _full((1, 8)), _full((8, 8))],
        out_specs=pl.BlockSpec((_RN, 3), lambda i: (i, 0)),
        out_shape=jax.ShapeDtypeStruct((_N, 3), f32),
    )(h, y, ct2, batch2, wd8, bd8, fac8)
    return out


# speedup vs baseline: 2.9461x; 1.1422x over previous
import functools

import jax
import jax.numpy as jnp
from jax import lax
from jax.experimental import pallas as pl
from jax.experimental.pallas import tpu as pltpu
from jax.experimental.pallas import tpu_sc as plsc

_N, _E, _B = 50000, 800000, 4
_NI, _PHI, _H = 7, 3, 64

_RN = 2000
_GN = _N // _RN
_RE = 2048
_EPAD = 802816
_GE = _EPAD // _RE
_CH = 128
_EPW = _EPAD // 32
_EPS = _EPAD // 16
_NACC = 51200
_RPT = _NACC // 16


def _stats_body(x_ref, b_ref, o_ref, acc_ref):
    i = pl.program_id(0)

    @pl.when(i == 0)
    def _():
        acc_ref[...] = jnp.zeros_like(acc_ref)

    xb = x_ref[...]
    bb = b_ref[...]
    feat = jnp.concatenate(
        [xb, jnp.ones((_RN, 1), jnp.float32), xb * xb,
         jnp.zeros((_RN, 1), jnp.float32)], axis=1)
    for b in range(_B):
        m = (bb == b).astype(jnp.float32)
        s = jnp.sum(feat * m, axis=0, keepdims=True)
        acc_ref[pl.ds(b, 1), :] = acc_ref[pl.ds(b, 1), :] + s

    @pl.when(i == _GN - 1)
    def _():
        A = acc_ref[...]
        cnt = jnp.maximum(A[0:4, 7:8], 1.0)
        mean = A[0:4, 0:7] / cnt
        ex2 = A[0:4, 8:15] / cnt
        var = ex2 - mean * mean
        gsum = jnp.sum(A[0:4, 3:7], axis=0, keepdims=True)
        gsq = jnp.sum(A[0:4, 11:15], axis=0, keepdims=True)
        gmean = gsum / float(_N)
        gvar = gsq / float(_N) - gmean * gmean
        sc_phi = 1.0 / (jnp.sqrt(jnp.maximum(var[:, 0:3], 0.0)) + 1e-8)
        sc_g = 1.0 / (jnp.sqrt(jnp.maximum(gvar, 0.0)) + 1e-8)
        sc_gb = jnp.broadcast_to(sc_g, (4, 4))
        sh_phi = -mean[:, 0:3] * sc_phi
        sh_g = jnp.broadcast_to(-gmean * sc_g, (4, 4))
        scale = jnp.concatenate(
            [sc_phi, sc_gb, jnp.ones((4, 1), jnp.float32)], axis=1)
        shift = jnp.concatenate(
            [sh_phi, sh_g, jnp.zeros((4, 1), jnp.float32)], axis=1)
        o_ref[...] = jnp.concatenate([scale, shift], axis=0)


def _sel4(bb, rows):
    return jnp.where(
        bb == 0, rows[0:1],
        jnp.where(bb == 1, rows[1:2],
                  jnp.where(bb == 2, rows[2:3], rows[3:4])))


def _encode_body(x_ref, c_ref, b_ref, st_ref, w_ref, bias_ref, xh_ref, h_ref):
    xb = x_ref[...]
    cb = c_ref[...]
    bb = b_ref[...]
    st = st_ref[...]
    sc = _sel4(bb, st[0:4])
    sh = _sel4(bb, st[4:8])
    xn = xb * sc[:, 0:7] + sh[:, 0:7]
    xn8 = jnp.concatenate([xn, jnp.zeros((_RN, 1), jnp.float32)], axis=1)
    h = jnp.maximum(
        jnp.dot(xn8, w_ref[...], preferred_element_type=jnp.float32)
        + bias_ref[...], 0.0)
    xh_ref[...] = jnp.concatenate(
        [xn, cb, jnp.zeros((_RN, 7), jnp.float32), h], axis=1)
    h_ref[...] = h


def _msg0_body(s_ref, r_ref, wee_ref, bee_ref, we_ref, ws_ref, wr_ref,
               bias_ref, enew_ref, mlo_ref, mhi_ref):
    sx = s_ref[...]
    rx = r_ref[...]
    d = sx[:, 0:16] - rx[:, 0:16]
    nrm = jnp.sqrt(d[:, 7:8] * d[:, 7:8] + d[:, 8:9] * d[:, 8:9])
    ea = jnp.concatenate(
        [d[:, 0:9], nrm, jnp.zeros((_RE, 6), jnp.float32)], axis=1)
    e = jnp.maximum(
        jnp.dot(ea, wee_ref[...], preferred_element_type=jnp.float32)
        + bee_ref[...], 0.0)
    m = (jnp.dot(e, we_ref[...], preferred_element_type=jnp.float32)
         + jnp.dot(sx[:, 16:80], ws_ref[...], preferred_element_type=jnp.float32)
         + jnp.dot(rx[:, 16:80], wr_ref[...], preferred_element_type=jnp.float32)
         + bias_ref[...])
    m = jnp.maximum(m, 0.0)
    enew_ref[...] = e + m
    mlo_ref[...] = m[:, 0:32]
    mhi_ref[...] = m[:, 32:64]


def _msg1_body(e_ref, hs_ref, hr_ref, we_ref, ws_ref, wr_ref, bias_ref,
               mlo_ref, mhi_ref):
    m = (jnp.dot(e_ref[...], we_ref[...], preferred_element_type=jnp.float32)
         + jnp.dot(hs_ref[...], ws_ref[...], preferred_element_type=jnp.float32)
         + jnp.dot(hr_ref[...], wr_ref[...], preferred_element_type=jnp.float32)
         + bias_ref[...])
    m = jnp.maximum(m, 0.0)
    mlo_ref[...] = m[:, 0:32]
    mhi_ref[...] = m[:, 32:64]


def _node_upd_body(h_ref, alo_ref, ahi_ref, wh_ref, wlo_ref, whi_ref,
                   bias_ref, out_ref):
    h = h_ref[...]
    u = (jnp.dot(h, wh_ref[...], preferred_element_type=jnp.float32)
         + jnp.dot(alo_ref[...], wlo_ref[...], preferred_element_type=jnp.float32)
         + jnp.dot(ahi_ref[...], whi_ref[...], preferred_element_type=jnp.float32)
         + bias_ref[...])
    out_ref[...] = h + jnp.maximum(u, 0.0)


def _upd_decode_body(h_ref, alo_ref, ahi_ref, wh_ref, wlo_ref, whi_ref,
                     bias_ref, y_ref, ct_ref, b_ref, wd_ref, bd_ref, f_ref,
                     out_ref):
    h = h_ref[...]
    u = (jnp.dot(h, wh_ref[...], preferred_element_type=jnp.float32)
         + jnp.dot(alo_ref[...], wlo_ref[...], preferred_element_type=jnp.float32)
         + jnp.dot(ahi_ref[...], whi_ref[...], preferred_element_type=jnp.float32)
         + bias_ref[...])
    h = h + jnp.maximum(u, 0.0)
    uvp = (jnp.dot(h, wd_ref[...], preferred_element_type=jnp.float32)
           + bd_ref[...])
    ct = ct_ref[...]
    bb = b_ref[...]
    mask_d = (ct == 6) | (ct == 4) | (ct == 7) | (ct == 8)
    mask_p = ct == 7
    yb = y_ref[...]
    uv = jnp.where(mask_d, yb[:, 0:2], uvp[:, 0:2])
    p = jnp.where(mask_p, 0.0, uvp[:, 2:3])
    fac = _sel4(bb, f_ref[...][0:4])
    out_ref[...] = jnp.concatenate([uv, p], axis=1) * fac[:, 0:3]


def _make_gather(d):
    mesh = plsc.VectorSubcoreMesh(core_axis_name="c", subcore_axis_name="s")

    @functools.partial(
        pl.kernel,
        out_type=[jax.ShapeDtypeStruct((_EPAD, d), jnp.float32),
                  jax.ShapeDtypeStruct((_EPAD, d), jnp.float32)],
        mesh=mesh,
        scratch_types=[pltpu.VMEM((_CH,), jnp.int32),
                       pltpu.VMEM((_CH,), jnp.int32),
                       pltpu.VMEM((_CH,), jnp.int32),
                       pltpu.VMEM((_CH,), jnp.int32),
                       pltpu.VMEM((_CH, d), jnp.float32),
                       pltpu.VMEM((_CH, d), jnp.float32),
                       pltpu.VMEM((_CH, d), jnp.float32),
                       pltpu.VMEM((_CH, d), jnp.float32)]
        + [pltpu.SemaphoreType.DMA] * 8,
        compiler_params=pltpu.CompilerParams(use_tc_tiling_on_sc=False),
    )
    def gat(tab, si, ri, out_s, out_r, ia0, ir0, ia1, ir1,
            ra0, rr0, ra1, rr1, g0, g1, g2, g3, t0, t1, t2, t3):
        wid = lax.axis_index("s") * 2 + lax.axis_index("c")
        base = wid * _EPW

        def body(j, carry):
            offa = base + 2 * j * _CH
            offb = offa + _CH
            pltpu.sync_copy(si.at[pl.ds(offa, _CH)], ia0)
            pltpu.sync_copy(ri.at[pl.ds(offa, _CH)], ir0)
            pltpu.sync_copy(si.at[pl.ds(offb, _CH)], ia1)
            pltpu.sync_copy(ri.at[pl.ds(offb, _CH)], ir1)
            ga0 = pltpu.async_copy(tab.at[ia0], ra0, g0)
            gr0 = pltpu.async_copy(tab.at[ir0], rr0, g1)
            ga1 = pltpu.async_copy(tab.at[ia1], ra1, g2)
            gr1 = pltpu.async_copy(tab.at[ir1], rr1, g3)
            ga0.wait()
            s0 = pltpu.async_copy(ra0, out_s.at[pl.ds(offa, _CH)], t0)
            gr0.wait()
            s1 = pltpu.async_copy(rr0, out_r.at[pl.ds(offa, _CH)], t1)
            ga1.wait()
            s2 = pltpu.async_copy(ra1, out_s.at[pl.ds(offb, _CH)], t2)
            gr1.wait()
            s3 = pltpu.async_copy(rr1, out_r.at[pl.ds(offb, _CH)], t3)
            s0.wait()
            s1.wait()
            s2.wait()
            s3.wait()
            return carry

        lax.fori_loop(0, _EPW // (2 * _CH), body, 0)

    return gat


_gather80 = _make_gather(80)
_gather64 = _make_gather(64)


_scatter_mesh = plsc.VectorSubcoreMesh(core_axis_name="c", subcore_axis_name="s")


@functools.partial(
    pl.kernel,
    out_type=[jax.ShapeDtypeStruct((_NACC, 32), jnp.float32),
              jax.ShapeDtypeStruct((_NACC, 32), jnp.float32)],
    mesh=_scatter_mesh,
    scratch_types=[pltpu.VMEM_SHARED((_NACC, 32), jnp.float32),
                   pltpu.VMEM((_CH,), jnp.int32),
                   pltpu.VMEM((_CH,), jnp.int32),
                   pltpu.VMEM((_CH, 32), jnp.float32),
                   pltpu.VMEM((_CH, 32), jnp.float32),
                   pltpu.VMEM((_CH, 32), jnp.float32)]
    + [pltpu.SemaphoreType.DMA] * 6,
    compiler_params=pltpu.CompilerParams(use_tc_tiling_on_sc=False),
)
def _scatter(ridx, mlo, mhi, alo, ahi, acc, i0, i1, m0, m1, zbuf,
             si0, si1, sm0, sm1, sa0, sa1):
    c = lax.axis_index("c")
    s = lax.axis_index("s")
    z16 = jnp.zeros((16,), jnp.float32)

    def zrow(i, carry):
        zbuf[i, pl.ds(0, 16)] = z16
        zbuf[i, pl.ds(16, 16)] = z16
        return carry

    lax.fori_loop(0, _CH, zrow, 0)
    rbase = s * _RPT

    def zc(k, carry):
        pltpu.sync_copy(zbuf, acc.at[pl.ds(rbase + k * _CH, _CH)])
        return carry

    lax.fori_loop(0, _RPT // _CH, zc, 0)
    plsc.subcore_barrier()

    ebase = s * _EPS

    def scatter_from(m_ref):
        def body(j, carry):
            offa = ebase + 2 * j * _CH
            offb = offa + _CH
            ci0 = pltpu.async_copy(ridx.at[pl.ds(offa, _CH)], i0, si0)
            cm0 = pltpu.async_copy(m_ref.at[pl.ds(offa, _CH)], m0, sm0)
            ci1 = pltpu.async_copy(ridx.at[pl.ds(offb, _CH)], i1, si1)
            cm1 = pltpu.async_copy(m_ref.at[pl.ds(offb, _CH)], m1, sm1)
            ci0.wait()
            cm0.wait()
            a0 = pltpu.async_copy(m0, acc.at[i0], sa0, add=True)
            ci1.wait()
            cm1.wait()
            a1 = pltpu.async_copy(m1, acc.at[i1], sa1, add=True)
            a0.wait()
            a1.wait()
            return carry

        lax.fori_loop(0, _EPS // (2 * _CH), body, 0)

    @pl.when(c == 0)
    def _():
        scatter_from(mlo)

    @pl.when(c == 1)
    def _():
        scatter_from(mhi)

    plsc.subcore_barrier()

    def wo(k, carry):
        r = rbase + k * _CH
        pltpu.sync_copy(acc.at[pl.ds(r, _CH)], m0)

        @pl.when(c == 0)
        def _():
            pltpu.sync_copy(m0, alo.at[pl.ds(r, _CH)])

        @pl.when(c == 1)
        def _():
            pltpu.sync_copy(m0, ahi.at[pl.ds(r, _CH)])

        return carry

    lax.fori_loop(0, _RPT // _CH, wo, 0)


def _full(shape):
    zeros = (0,) * len(shape)
    return pl.BlockSpec(shape, lambda i: zeros)


def kernel(x, cpd_centroid, y, uvp_dim, sigma, enc_nW, enc_nb, enc_eW, enc_eb,
           eW0, eb0, nW0, nb0, eW1, eb1, nW1, nb1, dec_W, dec_b,
           edge_index, batch, cell_type):
    f32 = jnp.float32
    batch2 = batch.reshape(_N, 1)
    ct2 = cell_type.reshape(_N, 1)
    s_idx = edge_index[0]
    r_idx = edge_index[1]
    pad = _EPAD - _E
    sg = jnp.concatenate([s_idx, jnp.zeros((pad,), jnp.int32)])
    rg = jnp.concatenate([r_idx, jnp.zeros((pad,), jnp.int32)])
    rs = jnp.concatenate([r_idx, jnp.full((pad,), _N, jnp.int32)])

    wn8 = jnp.concatenate([enc_nW, jnp.zeros((1, _H), f32)], axis=0)
    bn = enc_nb.reshape(1, _H)
    we16 = jnp.concatenate([enc_eW, jnp.zeros((6, _H), f32)], axis=0)
    be = enc_eb.reshape(1, _H)
    wd8 = jnp.concatenate([dec_W, jnp.zeros((_H, 5), f32)], axis=1)
    bd8 = jnp.concatenate([dec_b, jnp.zeros((5,), f32)]).reshape(1, 8)
    fac = uvp_dim * sigma
    fac8 = jnp.zeros((8, 8), f32).at[0:4, 0:3].set(fac)

    st = pl.pallas_call(
        _stats_body,
        grid=(_GN,),
        in_specs=[pl.BlockSpec((_RN, _NI), lambda i: (i, 0)),
                  pl.BlockSpec((_RN, 1), lambda i: (i, 0))],
        out_specs=_full((8, 8)),
        out_shape=jax.ShapeDtypeStruct((8, 8), f32),
        scratch_shapes=[pltpu.VMEM((8, 16), f32)],
    )(x, batch2)

    xh, h = pl.pallas_call(
        _encode_body,
        grid=(_GN,),
        in_specs=[pl.BlockSpec((_RN, _NI), lambda i: (i, 0)),
                  pl.BlockSpec((_RN, 2), lambda i: (i, 0)),
                  pl.BlockSpec((_RN, 1), lambda i: (i, 0)),
                  _full((8, 8)), _full((8, _H)), _full((1, _H))],
        out_specs=[pl.BlockSpec((_RN, 80), lambda i: (i, 0)),
                   pl.BlockSpec((_RN, _H), lambda i: (i, 0))],
        out_shape=[jax.ShapeDtypeStruct((_N, 80), f32),
                   jax.ShapeDtypeStruct((_N, _H), f32)],
    )(x, cpd_centroid, batch2, st, wn8, bn)

    xhs, xhr = _gather80(xh, sg, rg)
    e, mlo, mhi = pl.pallas_call(
        _msg0_body,
        grid=(_GE,),
        in_specs=[pl.BlockSpec((_RE, 80), lambda i: (i, 0)),
                  pl.BlockSpec((_RE, 80), lambda i: (i, 0)),
                  _full((16, _H)), _full((1, _H)),
                  _full((_H, _H)), _full((_H, _H)), _full((_H, _H)),
                  _full((1, _H))],
        out_specs=[pl.BlockSpec((_RE, _H), lambda i: (i, 0)),
                   pl.BlockSpec((_RE, 32), lambda i: (i, 0)),
                   pl.BlockSpec((_RE, 32), lambda i: (i, 0))],
        out_shape=[jax.ShapeDtypeStruct((_EPAD, _H), f32),
                   jax.ShapeDtypeStruct((_EPAD, 32), f32),
                   jax.ShapeDtypeStruct((_EPAD, 32), f32)],
    )(xhs, xhr, we16, be, eW0[0:_H], eW0[_H:2 * _H], eW0[2 * _H:3 * _H],
      eb0.reshape(1, _H))
    alo, ahi = _scatter(rs, mlo, mhi)
    h = pl.pallas_call(
        _node_upd_body,
        grid=(_GN,),
        in_specs=[pl.BlockSpec((_RN, _H), lambda i: (i, 0)),
                  pl.BlockSpec((_RN, 32), lambda i: (i, 0)),
                  pl.BlockSpec((_RN, 32), lambda i: (i, 0)),
                  _full((_H, _H)), _full((32, _H)), _full((32, _H)),
                  _full((1, _H))],
        out_specs=pl.BlockSpec((_RN, _H), lambda i: (i, 0)),
        out_shape=jax.ShapeDtypeStruct((_N, _H), f32),
    )(h, alo, ahi, nW0[0:_H], nW0[_H:_H + 32], nW0[_H + 32:_H + 64],
      nb0.reshape(1, _H))

    hs, hr = _gather64(h, sg, rg)
    mlo, mhi = pl.pallas_call(
        _msg1_body,
        grid=(_GE,),
        in_specs=[pl.BlockSpec((_RE, _H), lambda i: (i, 0)),
                  pl.BlockSpec((_RE, _H), lambda i: (i, 0)),
                  pl.BlockSpec((_RE, _H), lambda i: (i, 0)),
                  _full((_H, _H)), _full((_H, _H)), _full((_H, _H)),
                  _full((1, _H))],
        out_specs=[pl.BlockSpec((_RE, 32), lambda i: (i, 0)),
                   pl.BlockSpec((_RE, 32), lambda i: (i, 0))],
        out_shape=[jax.ShapeDtypeStruct((_EPAD, 32), f32),
                   jax.ShapeDtypeStruct((_EPAD, 32), f32)],
    )(e, hs, hr, eW1[0:_H], eW1[_H:2 * _H], eW1[2 * _H:3 * _H],
      eb1.reshape(1, _H))
    alo, ahi = _scatter(rs, mlo, mhi)

    out = pl.pallas_call(
        _upd_decode_body,
        grid=(_GN,),
        in_specs=[pl.BlockSpec((_RN, _H), lambda i: (i, 0)),
                  pl.BlockSpec((_RN, 32), lambda i: (i, 0)),
                  pl.BlockSpec((_RN, 32), lambda i: (i, 0)),
                  _full((_H, _H)), _full((32, _H)), _full((32, _H)),
                  _full((1, _H)),
                  pl.BlockSpec((_RN, 3), lambda i: (i, 0)),
                  pl.BlockSpec((_RN, 1), lambda i: (i, 0)),
                  pl.BlockSpec((_RN, 1), lambda i: (i, 0)),
                  _full((_H, 8)), _full((1, 8)), _full((8, 8))],
        out_specs=pl.BlockSpec((_RN, 3), lambda i: (i, 0)),
        out_shape=jax.ShapeDtypeStruct((_N, 3), f32),
    )(h, alo, ahi, nW1[0:_H], nW1[_H:_H + 32], nW1[_H + 32:_H + 64],
      nb1.reshape(1, _H), y, ct2, batch2, wd8, bd8, fac8)
    return out
